# baseline (speedup 1.0000x reference)
import functools

import jax
import jax.numpy as jnp
from jax import lax
from jax.experimental import pallas as pl
from jax.experimental.pallas import tpu as pltpu
from jax.experimental.pallas import tpu_sc as plsc

N = 10000
D = 128
G = 32
HEADS = 4
INNER = 64
KV = HEADS * INNER
LAMB = 0.5
SCALE = INNER ** -0.5

NP = 10240
EP = 337920
NWORK = 32
EPW = EP // NWORK
C = 96
NCHUNK = EPW // C
RPS = NP // 16
RPW = NP // NWORK
CB = 1024
FLUSH = 8192
STAGE = FLUSH + CB
BCAP = EP + FLUSH

@functools.cache
def _sc_degcount_kernel():
    mesh = plsc.VectorSubcoreMesh(core_axis_name="c", subcore_axis_name="s")
    return functools.partial(
        pl.kernel,
        mesh=mesh,
        out_type=jax.ShapeDtypeStruct((2, NP, 128), jnp.float32),
        compiler_params=pltpu.CompilerParams(needs_layout_passes=False),
        scratch_types=[
            pltpu.VMEM((C,), jnp.int32),
            pltpu.VMEM((C, 128), jnp.float32),
            pltpu.VMEM_SHARED((NP, 128), jnp.float32),
        ],
    )(_sc_degcount_body)


def _sc_degcount_body(pk_hbm, z128_hbm, out_hbm, row_v, ones_v, acc_sh):
    cid = lax.axis_index("c")
    sid = lax.axis_index("s")
    wid = sid * 2 + cid
    pltpu.sync_copy(z128_hbm.at[pl.ds(sid * RPS, RPS)],
                    acc_sh.at[pl.ds(sid * RPS, RPS)])

    def fill(i, carry):
        for j in range(8):
            ones_v[i, pl.ds(j * 16, 16)] = jnp.full((16,), 1.0, jnp.float32)
        return carry

    lax.fori_loop(0, C, fill, 0)
    plsc.subcore_barrier()

    def body(i, carry):
        off = pl.multiple_of(wid * EPW + i * C, 8)
        pltpu.sync_copy(pk_hbm.at[pl.ds(off, C)], row_v)
        for i16 in range(C // 16):
            pk = row_v[pl.ds(i16 * 16, 16)]
            row_v[pl.ds(i16 * 16, 16)] = lax.shift_right_logical(pk, 14)
        pltpu.sync_copy(ones_v, acc_sh.at[row_v], add=True)
        return carry

    lax.fori_loop(0, NCHUNK, body, 0)
    plsc.subcore_barrier()
    pltpu.sync_copy(acc_sh.at[pl.ds(sid * RPS, RPS)],
                    out_hbm.at[cid, pl.ds(sid * RPS, RPS)])


@functools.cache
def _sc_bucket_kernel():
    mesh = plsc.VectorSubcoreMesh(core_axis_name="c", subcore_axis_name="s")
    return functools.partial(
        pl.kernel,
        mesh=mesh,
        out_type=(jax.ShapeDtypeStruct((NWORK * BCAP,), jnp.int32),
                  jax.ShapeDtypeStruct((NWORK * 16,), jnp.int32)),
        compiler_params=pltpu.CompilerParams(needs_layout_passes=False),
        scratch_types=[
            pltpu.VMEM((CB,), jnp.int32),
            pltpu.VMEM((STAGE,), jnp.int32),
            pltpu.VMEM((16,), jnp.int32),
        ],
    )(_sc_bucket_body)


def _sc_bucket_body(pk_hbm, bk_hbm, cnt_hbm, pkb_v, stage_v, c16_v):
    cid = lax.axis_index("c")
    sid = lax.axis_index("s")
    wid = sid * 2 + cid
    lo = wid * RPW
    hi = lo + RPW

    def chunk(i, carry):
        ptr, off = carry
        pltpu.sync_copy(pk_hbm.at[pl.ds(pl.multiple_of(i * CB, 8), CB)], pkb_v)

        def sub(k, p):
            pk = pkb_v[pl.ds(k * 16, 16)]
            rv = lax.shift_right_logical(pk, 14)
            m = (rv >= lo) & (rv < hi)
            mi = m.astype(jnp.int32)
            incl = plsc.cumsum(mi)
            pos = p + incl - mi
            plsc.store_scatter(stage_v, [pos], pk, mask=m)
            return p + jnp.max(plsc.all_reduce_population_count(m))

        ptr = lax.fori_loop(0, CB // 16, sub, ptr)

        def do_flush(args):
            p, o = args
            pltpu.sync_copy(
                stage_v.at[pl.ds(0, FLUSH)],
                bk_hbm.at[pl.ds(pl.multiple_of(wid * BCAP + o, 8), FLUSH)])
            nmove = p - FLUSH

            def mv(k, carry2):
                @pl.when(k * 16 < nmove)
                def _():
                    stage_v[pl.ds(k * 16, 16)] = \
                        stage_v[pl.ds(FLUSH + k * 16, 16)]
                return carry2

            lax.fori_loop(0, CB // 16, mv, 0)
            return p - FLUSH, o + FLUSH

        ptr, off = lax.cond(ptr >= FLUSH, do_flush, lambda a: a, (ptr, off))
        return ptr, off

    ptr, off = lax.fori_loop(0, EP // CB, chunk,
                             (jnp.int32(0), jnp.int32(0)))
    pltpu.sync_copy(
        stage_v.at[pl.ds(0, FLUSH)],
        bk_hbm.at[pl.ds(pl.multiple_of(wid * BCAP + off, 8), FLUSH)])
    c16_v[pl.ds(0, 16)] = jnp.full((16,), 0, jnp.int32) + (off + ptr)
    pltpu.sync_copy(c16_v,
                    cnt_hbm.at[pl.ds(pl.multiple_of(wid * 16, 8), 16)])


@functools.cache
def _sc_aggregate_kernel():
    mesh = plsc.VectorSubcoreMesh(core_axis_name="c", subcore_axis_name="s")
    return functools.partial(
        pl.kernel,
        mesh=mesh,
        out_type=jax.ShapeDtypeStruct((NP, 128), jnp.float32),
        compiler_params=pltpu.CompilerParams(needs_layout_passes=False),
        scratch_types=[
            pltpu.VMEM((C,), jnp.int32),
            pltpu.VMEM((C,), jnp.int32),
            pltpu.VMEM((C,), jnp.int32),
            pltpu.VMEM((C,), jnp.float32),
            pltpu.VMEM((C, 128), jnp.float32),
            pltpu.VMEM((16,), jnp.int32),
            pltpu.VMEM((NP,), jnp.float32),
            pltpu.VMEM((NP,), jnp.float32),
            pltpu.VMEM((RPW, 128), jnp.float32),
            pltpu.SemaphoreType.DMA,
        ],
    )(_sc_aggregate_body)


def _sc_aggregate_body(xw_hbm, bk_hbm, cnt_hbm, dis_hbm, f_hbm, z128_hbm,
                       out_hbm, pk_v, col_v, row_v, w_v, rows_v, cnt_v,
                       dis_v, f_v, acc_v, sem):
    cid = lax.axis_index("c")
    sid = lax.axis_index("s")
    wid = sid * 2 + cid
    lo = wid * RPW
    pltpu.sync_copy(z128_hbm.at[pl.ds(0, RPW)], acc_v)
    pltpu.sync_copy(dis_hbm, dis_v)
    pltpu.sync_copy(f_hbm, f_v.at[pl.ds(0, N)])
    for t in range((NP - N) // 16):
        f_v[pl.ds(N + t * 16, 16)] = jnp.full((16,), 0.0, jnp.float32)
    pltpu.sync_copy(cnt_hbm.at[pl.ds(pl.multiple_of(wid * 16, 8), 16)], cnt_v)
    cnt = jnp.max(cnt_v[pl.ds(0, 16)])
    nch = (cnt + (C - 1)) // C

    def chunk(j, carry):
        off = pl.multiple_of(wid * BCAP + j * C, 8)
        pltpu.sync_copy(bk_hbm.at[pl.ds(off, C)], pk_v)
        for i16 in range(C // 16):
            s = pl.ds(i16 * 16, 16)
            g = j * C + i16 * 16 + lax.iota(jnp.int32, 16)
            valid = g < cnt
            pk = pk_v[s]
            rv = lax.shift_right_logical(pk, 14)
            cv = jnp.minimum(pk & 16383, N - 1)
            col_v[s] = cv
            row_v[s] = jnp.clip(rv - lo, 0, RPW - 1)
            rvc = jnp.minimum(rv, NP - 1)
            dr = plsc.load_gather(dis_v, [rvc])
            dc = plsc.load_gather(dis_v, [cv])
            fr = plsc.load_gather(f_v, [rvc])
            w_v[s] = jnp.where(valid, dr * dc + fr, 0.0)
        pltpu.async_copy(xw_hbm.at[col_v], rows_v, sem).wait()

        def acc_edge(e2, c2):
            for dd in range(2):
                ee = e2 * 2 + dd
                ws = plsc.load_gather(w_v, [jnp.full((16,), 0, jnp.int32) + ee])
                rl16 = plsc.load_gather(row_v,
                                        [jnp.full((16,), 0, jnp.int32) + ee])
                rls = jnp.max(rl16)
                for j8 in range(8):
                    sl = pl.ds(j8 * 16, 16)
                    acc_v[rls, sl] = acc_v[rls, sl] + rows_v[ee, sl] * ws
            return c2

        lax.fori_loop(0, C // 2, acc_edge, 0)
        return carry

    lax.fori_loop(0, nch, chunk, 0)
    pltpu.sync_copy(acc_v, out_hbm.at[pl.ds(pl.multiple_of(lo, 8), RPW)])


def _bn0_body(x_ref, g_ref, b_ref, X_ref):
    x = x_ref[...]
    m = jnp.mean(x, axis=0)
    v = jnp.mean((x - m) ** 2, axis=0)
    X_ref[...] = (x - m) * lax.rsqrt(v + 1e-5) * g_ref[...] + b_ref[...]


def _tc_bn0(x, g, b):
    return pl.pallas_call(
        _bn0_body,
        out_shape=jax.ShapeDtypeStruct((N, D), jnp.float32),
    )(x, g, b)


def _dis_body(deg_ref, dis_ref):
    deg = deg_ref[0, :, :16] + deg_ref[1, :, :16]
    dis_ref[...] = jnp.where(deg > 0.0, lax.rsqrt(jnp.maximum(deg, 1e-30)), 0.0)


def _tc_dis(deg2):
    return pl.pallas_call(
        _dis_body,
        out_shape=jax.ShapeDtypeStruct((NP, 16), jnp.float32),
    )(deg2)


def _bnl_body(xs_ref, g_ref, b_ref, X_ref):
    x = jnp.maximum(xs_ref[:N, :], 0.0)
    m = jnp.mean(x, axis=0)
    v = jnp.mean((x - m) ** 2, axis=0)
    X_ref[...] = (x - m) * lax.rsqrt(v + 1e-5) * g_ref[...] + b_ref[...]


def _tc_bnl(xs, g, b):
    return pl.pallas_call(
        _bnl_body,
        out_shape=jax.ShapeDtypeStruct((N, D), jnp.float32),
    )(xs, g, b)


def _cross_body(X_ref, batch_ref, M_ref, kw_ref, vw_ref, qw_ref, wo_ref,
                w1_ref, b1_ref, w2_ref, b2_ref, acc_ref, Mn_ref):
    h = pl.program_id(0)
    X = X_ref[...]
    k = X @ kw_ref[0]
    v = X @ vw_ref[0]
    qh = M_ref[...] @ qw_ref[0]
    sim = lax.dot_general(qh, k, (((1,), (1,)), ((), ()))) * SCALE
    g_iota = lax.broadcasted_iota(jnp.int32, (G, N), 0)
    mask = batch_ref[...] == g_iota
    sim = jnp.where(mask, sim, -1e9)
    mx = jnp.max(sim, axis=1, keepdims=True)
    e = jnp.exp(sim - mx)
    attn = e / jnp.sum(e, axis=1, keepdims=True)
    contrib = (attn @ v) @ wo_ref[0]

    @pl.when(h == 0)
    def _():
        acc_ref[...] = contrib

    @pl.when(h > 0)
    def _():
        acc_ref[...] = acc_ref[...] + contrib

    @pl.when(h == HEADS - 1)
    def _():
        M = M_ref[...] + acc_ref[...]
        M = M + jnp.maximum(M @ w1_ref[...] + b1_ref[...], 0.0) @ w2_ref[...] \
            + b2_ref[...]
        Mn_ref[...] = M


def _tc_cross(X, batch2d, M, tokv, p):
    full = lambda s: pl.BlockSpec(s, lambda h: (0,) * len(s))
    head = lambda s: pl.BlockSpec((1,) + s, lambda h: (h, 0, 0))
    kw4 = tokv[:, :KV].reshape(D, HEADS, INNER).transpose(1, 0, 2)
    vw4 = tokv[:, KV:].reshape(D, HEADS, INNER).transpose(1, 0, 2)
    qw4 = p['Wq'].reshape(D, HEADS, INNER).transpose(1, 0, 2)
    wo4 = p['Wo'].reshape(HEADS, INNER, D)
    acc, Mn = pl.pallas_call(
        _cross_body,
        grid=(HEADS,),
        in_specs=[full((N, D)), full((1, N)), full((G, D)),
                  head((D, INNER)), head((D, INNER)), head((D, INNER)),
                  head((INNER, D)), full((D, 2 * D)),
                  full((1, 2 * D)), full((2 * D, D)), full((1, D))],
        out_specs=(full((G, D)), full((G, D))),
        out_shape=(jax.ShapeDtypeStruct((G, D), jnp.float32),
                   jax.ShapeDtypeStruct((G, D), jnp.float32)),
    )(X, batch2d, M, kw4, vw4, qw4, wo4, p['W1'],
      p['b1'].reshape(1, -1), p['W2'], p['b2'].reshape(1, -1))
    return Mn


def _sim_body(X_ref, batch_ref, M_ref, wq_ref, wk_ref, gw_ref, gb_ref,
              f_ref, xw_ref):
    X = X_ref[...]
    k = X @ wk_ref[...]
    q = M_ref[...] @ wq_ref[...]
    sim = lax.dot_general(q, k, (((1,), (1,)), ((), ()))) * SCALE
    g_iota = lax.broadcasted_iota(jnp.int32, (G, N), 0)
    mask = batch_ref[...] == g_iota
    sim = jnp.where(mask, sim, -1e9)
    mx = jnp.max(sim, axis=1, keepdims=True)
    e = jnp.exp(sim - mx)
    attn = e / jnp.sum(e, axis=1, keepdims=True)
    f_ref[...] = jnp.sum(attn, axis=0, keepdims=True) * LAMB
    xw_ref[...] = X @ gw_ref[...] + gb_ref[...]


def _tc_sim(X, batch2d, M, wq, wk, gw, gb):
    return pl.pallas_call(
        _sim_body,
        out_shape=(jax.ShapeDtypeStruct((1, N), jnp.float32),
                   jax.ShapeDtypeStruct((N, D), jnp.float32)),
    )(X, batch2d, M, wq, wk, gw, gb.reshape(1, -1))


def _head_body(M_ref, w1_ref, b1_ref, w2_ref, b2_ref, out_ref):
    h = jnp.maximum(M_ref[...] @ w1_ref[...] + b1_ref[...], 0.0)
    out_ref[...] = h @ w2_ref[...] + b2_ref[...]


def _tc_head(M, w1, b1, w2, b2):
    nc = w2.shape[1]
    return pl.pallas_call(
        _head_body,
        out_shape=jax.ShapeDtypeStruct((G, nc), jnp.float32),
    )(M, w1, b1.reshape(1, -1), w2, b2.reshape(1, -1))


def kernel(x, edge_index, batch, num_graphs, params):
    p = params
    ei = edge_index.astype(jnp.int32)
    loops = jnp.arange(N, dtype=jnp.int32)
    pad = EP - (ei.shape[1] + N)
    row_p = jnp.concatenate([ei[0], loops, jnp.full((pad,), N, jnp.int32)])
    col_p = jnp.concatenate([ei[1], loops, jnp.zeros((pad,), jnp.int32)])
    pk_p = (row_p << 14) | col_p
    batch2d = batch.astype(jnp.int32).reshape(1, N)
    z128 = jnp.zeros((NP, 128), jnp.float32)

    deg2 = _sc_degcount_kernel()(pk_p, z128)
    bk, cnts = _sc_bucket_kernel()(pk_p)
    X = _tc_bn0(x, p['bn_feat_g'].reshape(1, -1), p['bn_feat_b'].reshape(1, -1))
    dis_flat = _tc_dis(deg2)[:, 0]

    M = jnp.tile(p['Memory'], (G, 1))
    for l in range(2):
        M = _tc_cross(X, batch2d, M, p['to_kv'][l], p)
        f1d, Xw = _tc_sim(X, batch2d, M, p['sim_q'], p['sim_k'][l],
                          p['gcn_W'][l], p['gcn_b'][l])
        xs = _sc_aggregate_kernel()(Xw, bk, cnts, dis_flat,
                                    f1d.reshape(-1), z128)
        X = _tc_bnl(xs, p['bn_g'][l].reshape(1, -1), p['bn_b'][l].reshape(1, -1))

    M = _tc_cross(X, batch2d, M, p['to_kv'][2], p)
    return _tc_head(M, p['fc1_W'], p['fc1_b'], p['fc2_W'], p['fc2_b'])


# speedup vs baseline: 2.8329x; 2.8329x over previous
import functools

import jax
import jax.numpy as jnp
from jax import lax
from jax.experimental import pallas as pl
from jax.experimental.pallas import tpu as pltpu
from jax.experimental.pallas import tpu_sc as plsc

N = 10000
D = 128
G = 32
HEADS = 4
INNER = 64
KV = HEADS * INNER
LAMB = 0.5
SCALE = INNER ** -0.5

NP = 10240
EP = 331776
NWORK = 32
EPW = EP // NWORK
C = 128
NCHUNK = EPW // C
RPS = NP // 16


@functools.cache
def _sc_degcount_kernel():
    mesh = plsc.VectorSubcoreMesh(core_axis_name="c", subcore_axis_name="s")
    return functools.partial(
        pl.kernel,
        mesh=mesh,
        out_type=jax.ShapeDtypeStruct((2, NP, 128), jnp.float32),
        compiler_params=pltpu.CompilerParams(needs_layout_passes=False),
        scratch_types=[
            pltpu.VMEM((C,), jnp.int32),
            pltpu.VMEM((C, 128), jnp.float32),
            pltpu.VMEM_SHARED((NP, 128), jnp.float32),
        ],
    )(_sc_degcount_body)


def _sc_degcount_body(row_hbm, z128_hbm, out_hbm, row_v, ones_v, acc_sh):
    cid = lax.axis_index("c")
    sid = lax.axis_index("s")
    wid = sid * 2 + cid
    pltpu.sync_copy(z128_hbm.at[pl.ds(sid * RPS, RPS)],
                    acc_sh.at[pl.ds(sid * RPS, RPS)])

    def fill(i, carry):
        for j in range(8):
            ones_v[i, pl.ds(j * 16, 16)] = jnp.full((16,), 1.0, jnp.float32)
        return carry

    lax.fori_loop(0, C, fill, 0)
    plsc.subcore_barrier()

    def body(i, carry):
        off = pl.multiple_of(wid * EPW + i * C, 8)
        pltpu.sync_copy(row_hbm.at[pl.ds(off, C)], row_v)
        pltpu.sync_copy(ones_v, acc_sh.at[row_v], add=True)
        return carry

    lax.fori_loop(0, NCHUNK, body, 0)
    plsc.subcore_barrier()
    pltpu.sync_copy(acc_sh.at[pl.ds(sid * RPS, RPS)],
                    out_hbm.at[cid, pl.ds(sid * RPS, RPS)])


@functools.cache
def _sc_aggregate_kernel():
    mesh = plsc.VectorSubcoreMesh(core_axis_name="c", subcore_axis_name="s")
    return functools.partial(
        pl.kernel,
        mesh=mesh,
        out_type=jax.ShapeDtypeStruct((2, NP, 128), jnp.float32),
        compiler_params=pltpu.CompilerParams(needs_layout_passes=False),
        scratch_types=[
            pltpu.VMEM((C,), jnp.int32),
            pltpu.VMEM((C,), jnp.int32),
            pltpu.VMEM((C,), jnp.float32),
            pltpu.VMEM((C, 128), jnp.float32),
            pltpu.VMEM((NP,), jnp.float32),
            pltpu.VMEM((NP,), jnp.float32),
            pltpu.VMEM_SHARED((NP, 128), jnp.float32),
            pltpu.SemaphoreType.DMA,
        ],
    )(_sc_aggregate_body)


def _sc_aggregate_body(xw_hbm, col_hbm, row_hbm, dis_hbm, f_hbm, z128_hbm, out_hbm,
                       col_v, row_v, w_v, rows_v, dis_v, f_v, acc_sh, sem):
    cid = lax.axis_index("c")
    sid = lax.axis_index("s")
    wid = sid * 2 + cid
    pltpu.sync_copy(z128_hbm.at[pl.ds(sid * RPS, RPS)],
                    acc_sh.at[pl.ds(sid * RPS, RPS)])
    pltpu.sync_copy(dis_hbm, dis_v)
    pltpu.sync_copy(f_hbm, f_v.at[pl.ds(0, N)])
    plsc.subcore_barrier()

    def body(i, carry):
        off = pl.multiple_of(wid * EPW + i * C, 8)
        pltpu.sync_copy(col_hbm.at[pl.ds(off, C)], col_v)
        pltpu.sync_copy(row_hbm.at[pl.ds(off, C)], row_v)
        pltpu.async_copy(xw_hbm.at[col_v], rows_v, sem).wait()
        for i16 in range(C // 16):
            rv = row_v[pl.ds(i16 * 16, 16)]
            cv = col_v[pl.ds(i16 * 16, 16)]
            dr = plsc.load_gather(dis_v, [rv])
            dc = plsc.load_gather(dis_v, [cv])
            fr = plsc.load_gather(f_v, [rv])
            w_v[pl.ds(i16 * 16, 16)] = dr * dc + fr

        def scale(e, c2):
            ws = plsc.load_gather(w_v, [jnp.full((16,), 0, jnp.int32) + e])
            for j in range(8):
                rows_v[e, pl.ds(j * 16, 16)] = rows_v[e, pl.ds(j * 16, 16)] * ws
            return c2

        lax.fori_loop(0, C, scale, 0)
        pltpu.sync_copy(rows_v, acc_sh.at[row_v], add=True)
        return carry

    lax.fori_loop(0, NCHUNK, body, 0)
    plsc.subcore_barrier()
    pltpu.sync_copy(acc_sh.at[pl.ds(sid * RPS, RPS)],
                    out_hbm.at[cid, pl.ds(sid * RPS, RPS)])


def _bn0_body(x_ref, g_ref, b_ref, X_ref):
    x = x_ref[...]
    m = jnp.mean(x, axis=0)
    v = jnp.mean((x - m) ** 2, axis=0)
    X_ref[...] = (x - m) * lax.rsqrt(v + 1e-5) * g_ref[...] + b_ref[...]


def _tc_bn0(x, g, b):
    return pl.pallas_call(
        _bn0_body,
        out_shape=jax.ShapeDtypeStruct((N, D), jnp.float32),
    )(x, g, b)


def _dis_body(deg_ref, dis_ref):
    deg = deg_ref[0, :, :16] + deg_ref[1, :, :16]
    dis_ref[...] = jnp.where(deg > 0.0, lax.rsqrt(jnp.maximum(deg, 1e-30)), 0.0)


def _tc_dis(deg2):
    return pl.pallas_call(
        _dis_body,
        out_shape=jax.ShapeDtypeStruct((NP, 16), jnp.float32),
    )(deg2)


def _bnl_body(xs_ref, g_ref, b_ref, X_ref):
    xsum = xs_ref[0, :N, :] + xs_ref[1, :N, :]
    x = jnp.maximum(xsum, 0.0)
    m = jnp.mean(x, axis=0)
    v = jnp.mean((x - m) ** 2, axis=0)
    X_ref[...] = (x - m) * lax.rsqrt(v + 1e-5) * g_ref[...] + b_ref[...]


def _tc_bnl(xs, g, b):
    return pl.pallas_call(
        _bnl_body,
        out_shape=jax.ShapeDtypeStruct((N, D), jnp.float32),
    )(xs, g, b)


def _cross_body(X_ref, batch_ref, M_ref, kw_ref, vw_ref, qw_ref, wo_ref,
                w1_ref, b1_ref, w2_ref, b2_ref, acc_ref, Mn_ref):
    h = pl.program_id(0)
    X = X_ref[...]
    k = X @ kw_ref[0]
    v = X @ vw_ref[0]
    qh = M_ref[...] @ qw_ref[0]
    sim = lax.dot_general(qh, k, (((1,), (1,)), ((), ()))) * SCALE
    g_iota = lax.broadcasted_iota(jnp.int32, (G, N), 0)
    mask = batch_ref[...] == g_iota
    sim = jnp.where(mask, sim, -1e9)
    mx = jnp.max(sim, axis=1, keepdims=True)
    e = jnp.exp(sim - mx)
    attn = e / jnp.sum(e, axis=1, keepdims=True)
    contrib = (attn @ v) @ wo_ref[0]

    @pl.when(h == 0)
    def _():
        acc_ref[...] = contrib

    @pl.when(h > 0)
    def _():
        acc_ref[...] = acc_ref[...] + contrib

    @pl.when(h == HEADS - 1)
    def _():
        M = M_ref[...] + acc_ref[...]
        M = M + jnp.maximum(M @ w1_ref[...] + b1_ref[...], 0.0) @ w2_ref[...] \
            + b2_ref[...]
        Mn_ref[...] = M


def _tc_cross(X, batch2d, M, tokv, p):
    full = lambda s: pl.BlockSpec(s, lambda h: (0,) * len(s))
    head = lambda s: pl.BlockSpec((1,) + s, lambda h: (h, 0, 0))
    kw4 = tokv[:, :KV].reshape(D, HEADS, INNER).transpose(1, 0, 2)
    vw4 = tokv[:, KV:].reshape(D, HEADS, INNER).transpose(1, 0, 2)
    qw4 = p['Wq'].reshape(D, HEADS, INNER).transpose(1, 0, 2)
    wo4 = p['Wo'].reshape(HEADS, INNER, D)
    acc, Mn = pl.pallas_call(
        _cross_body,
        grid=(HEADS,),
        in_specs=[full((N, D)), full((1, N)), full((G, D)),
                  head((D, INNER)), head((D, INNER)), head((D, INNER)),
                  head((INNER, D)), full((D, 2 * D)),
                  full((1, 2 * D)), full((2 * D, D)), full((1, D))],
        out_specs=(full((G, D)), full((G, D))),
        out_shape=(jax.ShapeDtypeStruct((G, D), jnp.float32),
                   jax.ShapeDtypeStruct((G, D), jnp.float32)),
    )(X, batch2d, M, kw4, vw4, qw4, wo4, p['W1'],
      p['b1'].reshape(1, -1), p['W2'], p['b2'].reshape(1, -1))
    return Mn


def _sim_body(X_ref, batch_ref, M_ref, wq_ref, wk_ref, gw_ref, gb_ref,
              f_ref, xw_ref):
    X = X_ref[...]
    k = X @ wk_ref[...]
    q = M_ref[...] @ wq_ref[...]
    sim = lax.dot_general(q, k, (((1,), (1,)), ((), ()))) * SCALE
    g_iota = lax.broadcasted_iota(jnp.int32, (G, N), 0)
    mask = batch_ref[...] == g_iota
    sim = jnp.where(mask, sim, -1e9)
    mx = jnp.max(sim, axis=1, keepdims=True)
    e = jnp.exp(sim - mx)
    attn = e / jnp.sum(e, axis=1, keepdims=True)
    f_ref[...] = jnp.sum(attn, axis=0, keepdims=True) * LAMB
    xw_ref[...] = X @ gw_ref[...] + gb_ref[...]


def _tc_sim(X, batch2d, M, wq, wk, gw, gb):
    return pl.pallas_call(
        _sim_body,
        out_shape=(jax.ShapeDtypeStruct((1, N), jnp.float32),
                   jax.ShapeDtypeStruct((N, D), jnp.float32)),
    )(X, batch2d, M, wq, wk, gw, gb.reshape(1, -1))


def _head_body(M_ref, w1_ref, b1_ref, w2_ref, b2_ref, out_ref):
    h = jnp.maximum(M_ref[...] @ w1_ref[...] + b1_ref[...], 0.0)
    out_ref[...] = h @ w2_ref[...] + b2_ref[...]


def _tc_head(M, w1, b1, w2, b2):
    nc = w2.shape[1]
    return pl.pallas_call(
        _head_body,
        out_shape=jax.ShapeDtypeStruct((G, nc), jnp.float32),
    )(M, w1, b1.reshape(1, -1), w2, b2.reshape(1, -1))


def kernel(x, edge_index, batch, num_graphs, params):
    p = params
    ei = edge_index.astype(jnp.int32)
    loops = jnp.arange(N, dtype=jnp.int32)
    pad = EP - (ei.shape[1] + N)
    row_p = jnp.concatenate([ei[0], loops, jnp.full((pad,), N, jnp.int32)])
    col_p = jnp.concatenate([ei[1], loops, jnp.zeros((pad,), jnp.int32)])
    batch2d = batch.astype(jnp.int32).reshape(1, N)
    z128 = jnp.zeros((NP, 128), jnp.float32)

    deg2 = _sc_degcount_kernel()(row_p, z128)
    X = _tc_bn0(x, p['bn_feat_g'].reshape(1, -1), p['bn_feat_b'].reshape(1, -1))
    dis_flat = _tc_dis(deg2)[:, 0]

    M = jnp.tile(p['Memory'], (G, 1))
    for l in range(2):
        M = _tc_cross(X, batch2d, M, p['to_kv'][l], p)
        f1d, Xw = _tc_sim(X, batch2d, M, p['sim_q'], p['sim_k'][l],
                          p['gcn_W'][l], p['gcn_b'][l])
        xs = _sc_aggregate_kernel()(Xw, col_p, row_p, dis_flat,
                                    f1d.reshape(-1), z128)
        X = _tc_bnl(xs, p['bn_g'][l].reshape(1, -1), p['bn_b'][l].reshape(1, -1))

    M = _tc_cross(X, batch2d, M, p['to_kv'][2], p)
    return _tc_head(M, p['fc1_W'], p['fc1_b'], p['fc2_W'], p['fc2_b'])


# speedup vs baseline: 3.0265x; 1.0683x over previous
import functools

import jax
import jax.numpy as jnp
from jax import lax
from jax.experimental import pallas as pl
from jax.experimental.pallas import tpu as pltpu
from jax.experimental.pallas import tpu_sc as plsc

N = 10000
D = 128
G = 32
HEADS = 4
INNER = 64
KV = HEADS * INNER
LAMB = 0.5
SCALE = INNER ** -0.5

NP = 10240
EP = 331776
NWORK = 32
EPW = EP // NWORK
C = 128
NCHUNK = EPW // C
RPS = NP // 16


@functools.cache
def _sc_degcount_kernel():
    mesh = plsc.VectorSubcoreMesh(core_axis_name="c", subcore_axis_name="s")
    return functools.partial(
        pl.kernel,
        mesh=mesh,
        out_type=jax.ShapeDtypeStruct((2, NP, 128), jnp.float32),
        compiler_params=pltpu.CompilerParams(needs_layout_passes=False),
        scratch_types=[
            pltpu.VMEM((C,), jnp.int32),
            pltpu.VMEM((C, 128), jnp.float32),
            pltpu.VMEM_SHARED((NP, 128), jnp.float32),
        ],
    )(_sc_degcount_body)


def _sc_degcount_body(row_hbm, z128_hbm, out_hbm, row_v, ones_v, acc_sh):
    cid = lax.axis_index("c")
    sid = lax.axis_index("s")
    wid = sid * 2 + cid
    pltpu.sync_copy(z128_hbm.at[pl.ds(sid * RPS, RPS)],
                    acc_sh.at[pl.ds(sid * RPS, RPS)])

    def fill(i, carry):
        for j in range(8):
            ones_v[i, pl.ds(j * 16, 16)] = jnp.full((16,), 1.0, jnp.float32)
        return carry

    lax.fori_loop(0, C, fill, 0)
    plsc.subcore_barrier()

    def body(i, carry):
        off = pl.multiple_of(wid * EPW + i * C, 8)
        pltpu.sync_copy(row_hbm.at[pl.ds(off, C)], row_v)
        pltpu.sync_copy(ones_v, acc_sh.at[row_v], add=True)
        return carry

    lax.fori_loop(0, NCHUNK, body, 0)
    plsc.subcore_barrier()
    pltpu.sync_copy(acc_sh.at[pl.ds(sid * RPS, RPS)],
                    out_hbm.at[cid, pl.ds(sid * RPS, RPS)])


@functools.cache
def _sc_aggregate_kernel():
    mesh = plsc.VectorSubcoreMesh(core_axis_name="c", subcore_axis_name="s")
    return functools.partial(
        pl.kernel,
        mesh=mesh,
        out_type=jax.ShapeDtypeStruct((2, NP, 128), jnp.float32),
        compiler_params=pltpu.CompilerParams(needs_layout_passes=False),
        scratch_types=[
            pltpu.VMEM((C,), jnp.int32),
            pltpu.VMEM((C,), jnp.int32),
            pltpu.VMEM((C,), jnp.float32),
            pltpu.VMEM((C, 128), jnp.float32),
            pltpu.VMEM((NP,), jnp.float32),
            pltpu.VMEM((NP,), jnp.float32),
            pltpu.VMEM_SHARED((NP, 128), jnp.float32),
            pltpu.SemaphoreType.DMA,
        ],
    )(_sc_aggregate_body)


def _sc_aggregate_body(xw_hbm, col_hbm, row_hbm, dis_hbm, f_hbm, z128_hbm, out_hbm,
                       col_v, row_v, w_v, rows_v, dis_v, f_v, acc_sh, sem):
    cid = lax.axis_index("c")
    sid = lax.axis_index("s")
    wid = sid * 2 + cid
    pltpu.sync_copy(z128_hbm.at[pl.ds(sid * RPS, RPS)],
                    acc_sh.at[pl.ds(sid * RPS, RPS)])
    pltpu.sync_copy(dis_hbm, dis_v)
    pltpu.sync_copy(f_hbm, f_v.at[pl.ds(0, N)])
    plsc.subcore_barrier()

    def body(i, carry):
        off = pl.multiple_of(wid * EPW + i * C, 8)
        pltpu.sync_copy(col_hbm.at[pl.ds(off, C)], col_v)
        pltpu.sync_copy(row_hbm.at[pl.ds(off, C)], row_v)
        pltpu.async_copy(xw_hbm.at[col_v], rows_v, sem).wait()
        for i16 in range(C // 16):
            rv = row_v[pl.ds(i16 * 16, 16)]
            cv = col_v[pl.ds(i16 * 16, 16)]
            dr = plsc.load_gather(dis_v, [rv])
            dc = plsc.load_gather(dis_v, [cv])
            fr = plsc.load_gather(f_v, [rv])
            w_v[pl.ds(i16 * 16, 16)] = dr * dc + fr

        def scale(e2, c2):
            e = e2 * 2
            ws0 = plsc.load_gather(w_v, [jnp.full((16,), 0, jnp.int32) + e])
            ws1 = plsc.load_gather(w_v, [jnp.full((16,), 1, jnp.int32) + e])
            for j in range(8):
                rows_v[e, pl.ds(j * 16, 16)] = rows_v[e, pl.ds(j * 16, 16)] * ws0
                rows_v[e + 1, pl.ds(j * 16, 16)] = \
                    rows_v[e + 1, pl.ds(j * 16, 16)] * ws1
            return c2

        lax.fori_loop(0, C // 2, scale, 0)
        pltpu.sync_copy(rows_v, acc_sh.at[row_v], add=True)
        return carry

    lax.fori_loop(0, NCHUNK, body, 0)
    plsc.subcore_barrier()
    pltpu.sync_copy(acc_sh.at[pl.ds(sid * RPS, RPS)],
                    out_hbm.at[cid, pl.ds(sid * RPS, RPS)])


def _bn0_body(x_ref, g_ref, b_ref, X_ref):
    x = x_ref[...]
    m = jnp.mean(x, axis=0)
    v = jnp.mean((x - m) ** 2, axis=0)
    X_ref[...] = (x - m) * lax.rsqrt(v + 1e-5) * g_ref[...] + b_ref[...]


def _tc_bn0(x, g, b):
    return pl.pallas_call(
        _bn0_body,
        out_shape=jax.ShapeDtypeStruct((N, D), jnp.float32),
    )(x, g, b)


def _dis_body(deg_ref, dis_ref):
    deg = deg_ref[0, :, :16] + deg_ref[1, :, :16]
    dis_ref[...] = jnp.where(deg > 0.0, lax.rsqrt(jnp.maximum(deg, 1e-30)), 0.0)


def _tc_dis(deg2):
    return pl.pallas_call(
        _dis_body,
        out_shape=jax.ShapeDtypeStruct((NP, 16), jnp.float32),
    )(deg2)


def _bnl_body(xs_ref, g_ref, b_ref, X_ref):
    xsum = xs_ref[0, :N, :] + xs_ref[1, :N, :]
    x = jnp.maximum(xsum, 0.0)
    m = jnp.mean(x, axis=0)
    v = jnp.mean((x - m) ** 2, axis=0)
    X_ref[...] = (x - m) * lax.rsqrt(v + 1e-5) * g_ref[...] + b_ref[...]


def _tc_bnl(xs, g, b):
    return pl.pallas_call(
        _bnl_body,
        out_shape=jax.ShapeDtypeStruct((N, D), jnp.float32),
    )(xs, g, b)


def _cross_body(X_ref, batch_ref, M_ref, kw_ref, vw_ref, qw_ref, wo_ref,
                w1_ref, b1_ref, w2_ref, b2_ref, acc_ref, Mn_ref):
    h = pl.program_id(0)
    X = X_ref[...]
    k = X @ kw_ref[0]
    v = X @ vw_ref[0]
    qh = M_ref[...] @ qw_ref[0]
    sim = lax.dot_general(qh, k, (((1,), (1,)), ((), ()))) * SCALE
    g_iota = lax.broadcasted_iota(jnp.int32, (G, N), 0)
    mask = batch_ref[...] == g_iota
    sim = jnp.where(mask, sim, -1e9)
    mx = jnp.max(sim, axis=1, keepdims=True)
    e = jnp.exp(sim - mx)
    attn = e / jnp.sum(e, axis=1, keepdims=True)
    contrib = (attn @ v) @ wo_ref[0]

    @pl.when(h == 0)
    def _():
        acc_ref[...] = contrib

    @pl.when(h > 0)
    def _():
        acc_ref[...] = acc_ref[...] + contrib

    @pl.when(h == HEADS - 1)
    def _():
        M = M_ref[...] + acc_ref[...]
        M = M + jnp.maximum(M @ w1_ref[...] + b1_ref[...], 0.0) @ w2_ref[...] \
            + b2_ref[...]
        Mn_ref[...] = M


def _tc_cross(X, batch2d, M, tokv, p):
    full = lambda s: pl.BlockSpec(s, lambda h: (0,) * len(s))
    head = lambda s: pl.BlockSpec((1,) + s, lambda h: (h, 0, 0))
    kw4 = tokv[:, :KV].reshape(D, HEADS, INNER).transpose(1, 0, 2)
    vw4 = tokv[:, KV:].reshape(D, HEADS, INNER).transpose(1, 0, 2)
    qw4 = p['Wq'].reshape(D, HEADS, INNER).transpose(1, 0, 2)
    wo4 = p['Wo'].reshape(HEADS, INNER, D)
    acc, Mn = pl.pallas_call(
        _cross_body,
        grid=(HEADS,),
        in_specs=[full((N, D)), full((1, N)), full((G, D)),
                  head((D, INNER)), head((D, INNER)), head((D, INNER)),
                  head((INNER, D)), full((D, 2 * D)),
                  full((1, 2 * D)), full((2 * D, D)), full((1, D))],
        out_specs=(full((G, D)), full((G, D))),
        out_shape=(jax.ShapeDtypeStruct((G, D), jnp.float32),
                   jax.ShapeDtypeStruct((G, D), jnp.float32)),
    )(X, batch2d, M, kw4, vw4, qw4, wo4, p['W1'],
      p['b1'].reshape(1, -1), p['W2'], p['b2'].reshape(1, -1))
    return Mn


def _sim_body(X_ref, batch_ref, M_ref, wq_ref, wk_ref, gw_ref, gb_ref,
              f_ref, xw_ref):
    X = X_ref[...]
    k = X @ wk_ref[...]
    q = M_ref[...] @ wq_ref[...]
    sim = lax.dot_general(q, k, (((1,), (1,)), ((), ()))) * SCALE
    g_iota = lax.broadcasted_iota(jnp.int32, (G, N), 0)
    mask = batch_ref[...] == g_iota
    sim = jnp.where(mask, sim, -1e9)
    mx = jnp.max(sim, axis=1, keepdims=True)
    e = jnp.exp(sim - mx)
    attn = e / jnp.sum(e, axis=1, keepdims=True)
    f_ref[...] = jnp.sum(attn, axis=0, keepdims=True) * LAMB
    xw_ref[...] = X @ gw_ref[...] + gb_ref[...]


def _tc_sim(X, batch2d, M, wq, wk, gw, gb):
    return pl.pallas_call(
        _sim_body,
        out_shape=(jax.ShapeDtypeStruct((1, N), jnp.float32),
                   jax.ShapeDtypeStruct((N, D), jnp.float32)),
    )(X, batch2d, M, wq, wk, gw, gb.reshape(1, -1))


def _head_body(M_ref, w1_ref, b1_ref, w2_ref, b2_ref, out_ref):
    h = jnp.maximum(M_ref[...] @ w1_ref[...] + b1_ref[...], 0.0)
    out_ref[...] = h @ w2_ref[...] + b2_ref[...]


def _tc_head(M, w1, b1, w2, b2):
    nc = w2.shape[1]
    return pl.pallas_call(
        _head_body,
        out_shape=jax.ShapeDtypeStruct((G, nc), jnp.float32),
    )(M, w1, b1.reshape(1, -1), w2, b2.reshape(1, -1))


def kernel(x, edge_index, batch, num_graphs, params):
    p = params
    ei = edge_index.astype(jnp.int32)
    loops = jnp.arange(N, dtype=jnp.int32)
    pad = EP - (ei.shape[1] + N)
    row_p = jnp.concatenate([ei[0], loops, jnp.full((pad,), N, jnp.int32)])
    col_p = jnp.concatenate([ei[1], loops, jnp.zeros((pad,), jnp.int32)])
    batch2d = batch.astype(jnp.int32).reshape(1, N)
    z128 = jnp.zeros((NP, 128), jnp.float32)

    deg2 = _sc_degcount_kernel()(row_p, z128)
    X = _tc_bn0(x, p['bn_feat_g'].reshape(1, -1), p['bn_feat_b'].reshape(1, -1))
    dis_flat = _tc_dis(deg2)[:, 0]

    M = jnp.tile(p['Memory'], (G, 1))
    for l in range(2):
        M = _tc_cross(X, batch2d, M, p['to_kv'][l], p)
        f1d, Xw = _tc_sim(X, batch2d, M, p['sim_q'], p['sim_k'][l],
                          p['gcn_W'][l], p['gcn_b'][l])
        xs = _sc_aggregate_kernel()(Xw, col_p, row_p, dis_flat,
                                    f1d.reshape(-1), z128)
        X = _tc_bnl(xs, p['bn_g'][l].reshape(1, -1), p['bn_b'][l].reshape(1, -1))

    M = _tc_cross(X, batch2d, M, p['to_kv'][2], p)
    return _tc_head(M, p['fc1_W'], p['fc1_b'], p['fc2_W'], p['fc2_b'])


# speedup vs baseline: 3.0953x; 1.0227x over previous
import functools

import jax
import jax.numpy as jnp
from jax import lax
from jax.experimental import pallas as pl
from jax.experimental.pallas import tpu as pltpu
from jax.experimental.pallas import tpu_sc as plsc

N = 10000
D = 128
G = 32
HEADS = 4
INNER = 64
KV = HEADS * INNER
LAMB = 0.5
SCALE = INNER ** -0.5

NP = 10240
EP = 331776
NWORK = 32
EPW = EP // NWORK
C = 128
NCHUNK = EPW // C
RPS = NP // 16


@functools.cache
def _sc_degcount_kernel():
    mesh = plsc.VectorSubcoreMesh(core_axis_name="c", subcore_axis_name="s")
    return functools.partial(
        pl.kernel,
        mesh=mesh,
        out_type=jax.ShapeDtypeStruct((2, NP, 128), jnp.float32),
        compiler_params=pltpu.CompilerParams(needs_layout_passes=False),
        scratch_types=[
            pltpu.VMEM((C,), jnp.int32),
            pltpu.VMEM((C, 128), jnp.float32),
            pltpu.VMEM_SHARED((NP, 128), jnp.float32),
        ],
    )(_sc_degcount_body)


def _sc_degcount_body(row_hbm, z128_hbm, out_hbm, row_v, ones_v, acc_sh):
    cid = lax.axis_index("c")
    sid = lax.axis_index("s")
    wid = sid * 2 + cid
    pltpu.sync_copy(z128_hbm.at[pl.ds(sid * RPS, RPS)],
                    acc_sh.at[pl.ds(sid * RPS, RPS)])

    def fill(i, carry):
        for j in range(8):
            ones_v[i, pl.ds(j * 16, 16)] = jnp.full((16,), 1.0, jnp.float32)
        return carry

    lax.fori_loop(0, C, fill, 0)
    plsc.subcore_barrier()

    def body(i, carry):
        off = pl.multiple_of(wid * EPW + i * C, 8)
        pltpu.sync_copy(row_hbm.at[pl.ds(off, C)], row_v)
        pltpu.sync_copy(ones_v, acc_sh.at[row_v], add=True)
        return carry

    lax.fori_loop(0, NCHUNK, body, 0)
    plsc.subcore_barrier()
    pltpu.sync_copy(acc_sh.at[pl.ds(sid * RPS, RPS)],
                    out_hbm.at[cid, pl.ds(sid * RPS, RPS)])


@functools.cache
def _sc_aggregate_kernel():
    mesh = plsc.VectorSubcoreMesh(core_axis_name="c", subcore_axis_name="s")
    return functools.partial(
        pl.kernel,
        mesh=mesh,
        out_type=jax.ShapeDtypeStruct((2, NP, 128), jnp.float32),
        compiler_params=pltpu.CompilerParams(needs_layout_passes=False),
        scratch_types=[
            pltpu.VMEM((C,), jnp.int32),
            pltpu.VMEM((C,), jnp.int32),
            pltpu.VMEM((C,), jnp.float32),
            pltpu.VMEM((C, 128), jnp.float32),
            pltpu.VMEM((NP,), jnp.float32),
            pltpu.VMEM((NP,), jnp.float32),
            pltpu.VMEM_SHARED((NP, 128), jnp.float32),
            pltpu.SemaphoreType.DMA,
        ],
    )(_sc_aggregate_body)


def _sc_aggregate_body(xw_hbm, col_hbm, row_hbm, dis_hbm, f_hbm, z128_hbm, out_hbm,
                       col_v, row_v, w_v, rows_v, dis_v, f_v, acc_sh, sem):
    cid = lax.axis_index("c")
    sid = lax.axis_index("s")
    wid = sid * 2 + cid
    pltpu.sync_copy(z128_hbm.at[pl.ds(sid * RPS, RPS)],
                    acc_sh.at[pl.ds(sid * RPS, RPS)])
    pltpu.sync_copy(dis_hbm, dis_v)
    pltpu.sync_copy(f_hbm, f_v.at[pl.ds(0, N)])
    plsc.subcore_barrier()

    def body(i, carry):
        off = pl.multiple_of(wid * EPW + i * C, 8)
        pltpu.sync_copy(col_hbm.at[pl.ds(off, C)], col_v)
        pltpu.sync_copy(row_hbm.at[pl.ds(off, C)], row_v)
        pltpu.async_copy(xw_hbm.at[col_v], rows_v, sem).wait()
        for i16 in range(C // 16):
            rv = row_v[pl.ds(i16 * 16, 16)]
            cv = col_v[pl.ds(i16 * 16, 16)]
            dr = plsc.load_gather(dis_v, [rv])
            dc = plsc.load_gather(dis_v, [cv])
            fr = plsc.load_gather(f_v, [rv])
            w_v[pl.ds(i16 * 16, 16)] = dr * dc + fr

        def scale(e4, c2):
            e = e4 * 4
            ws = [plsc.load_gather(w_v, [jnp.full((16,), d, jnp.int32) + e])
                  for d in range(4)]
            for j in range(8):
                for d in range(4):
                    rows_v[e + d, pl.ds(j * 16, 16)] = \
                        rows_v[e + d, pl.ds(j * 16, 16)] * ws[d]
            return c2

        lax.fori_loop(0, C // 4, scale, 0)
        pltpu.sync_copy(rows_v, acc_sh.at[row_v], add=True)
        return carry

    lax.fori_loop(0, NCHUNK, body, 0)
    plsc.subcore_barrier()
    pltpu.sync_copy(acc_sh.at[pl.ds(sid * RPS, RPS)],
                    out_hbm.at[cid, pl.ds(sid * RPS, RPS)])


def _bn0_body(x_ref, g_ref, b_ref, X_ref):
    x = x_ref[...]
    m = jnp.mean(x, axis=0)
    v = jnp.mean((x - m) ** 2, axis=0)
    X_ref[...] = (x - m) * lax.rsqrt(v + 1e-5) * g_ref[...] + b_ref[...]


def _tc_bn0(x, g, b):
    return pl.pallas_call(
        _bn0_body,
        out_shape=jax.ShapeDtypeStruct((N, D), jnp.float32),
    )(x, g, b)


def _dis_body(deg_ref, dis_ref):
    deg = deg_ref[0, :, :16] + deg_ref[1, :, :16]
    dis_ref[...] = jnp.where(deg > 0.0, lax.rsqrt(jnp.maximum(deg, 1e-30)), 0.0)


def _tc_dis(deg2):
    return pl.pallas_call(
        _dis_body,
        out_shape=jax.ShapeDtypeStruct((NP, 16), jnp.float32),
    )(deg2)


def _bnl_body(xs_ref, g_ref, b_ref, X_ref):
    xsum = xs_ref[0, :N, :] + xs_ref[1, :N, :]
    x = jnp.maximum(xsum, 0.0)
    m = jnp.mean(x, axis=0)
    v = jnp.mean((x - m) ** 2, axis=0)
    X_ref[...] = (x - m) * lax.rsqrt(v + 1e-5) * g_ref[...] + b_ref[...]


def _tc_bnl(xs, g, b):
    return pl.pallas_call(
        _bnl_body,
        out_shape=jax.ShapeDtypeStruct((N, D), jnp.float32),
    )(xs, g, b)


def _cross_body(X_ref, batch_ref, M_ref, kw_ref, vw_ref, qw_ref, wo_ref,
                w1_ref, b1_ref, w2_ref, b2_ref, acc_ref, Mn_ref):
    h = pl.program_id(0)
    X = X_ref[...]
    k = X @ kw_ref[0]
    v = X @ vw_ref[0]
    qh = M_ref[...] @ qw_ref[0]
    sim = lax.dot_general(qh, k, (((1,), (1,)), ((), ()))) * SCALE
    g_iota = lax.broadcasted_iota(jnp.int32, (G, N), 0)
    mask = batch_ref[...] == g_iota
    sim = jnp.where(mask, sim, -1e9)
    mx = jnp.max(sim, axis=1, keepdims=True)
    e = jnp.exp(sim - mx)
    attn = e / jnp.sum(e, axis=1, keepdims=True)
    contrib = (attn @ v) @ wo_ref[0]

    @pl.when(h == 0)
    def _():
        acc_ref[...] = contrib

    @pl.when(h > 0)
    def _():
        acc_ref[...] = acc_ref[...] + contrib

    @pl.when(h == HEADS - 1)
    def _():
        M = M_ref[...] + acc_ref[...]
        M = M + jnp.maximum(M @ w1_ref[...] + b1_ref[...], 0.0) @ w2_ref[...] \
            + b2_ref[...]
        Mn_ref[...] = M


def _tc_cross(X, batch2d, M, tokv, p):
    full = lambda s: pl.BlockSpec(s, lambda h: (0,) * len(s))
    head = lambda s: pl.BlockSpec((1,) + s, lambda h: (h, 0, 0))
    kw4 = tokv[:, :KV].reshape(D, HEADS, INNER).transpose(1, 0, 2)
    vw4 = tokv[:, KV:].reshape(D, HEADS, INNER).transpose(1, 0, 2)
    qw4 = p['Wq'].reshape(D, HEADS, INNER).transpose(1, 0, 2)
    wo4 = p['Wo'].reshape(HEADS, INNER, D)
    acc, Mn = pl.pallas_call(
        _cross_body,
        grid=(HEADS,),
        in_specs=[full((N, D)), full((1, N)), full((G, D)),
                  head((D, INNER)), head((D, INNER)), head((D, INNER)),
                  head((INNER, D)), full((D, 2 * D)),
                  full((1, 2 * D)), full((2 * D, D)), full((1, D))],
        out_specs=(full((G, D)), full((G, D))),
        out_shape=(jax.ShapeDtypeStruct((G, D), jnp.float32),
                   jax.ShapeDtypeStruct((G, D), jnp.float32)),
    )(X, batch2d, M, kw4, vw4, qw4, wo4, p['W1'],
      p['b1'].reshape(1, -1), p['W2'], p['b2'].reshape(1, -1))
    return Mn


def _sim_body(X_ref, batch_ref, M_ref, wq_ref, wk_ref, gw_ref, gb_ref,
              f_ref, xw_ref):
    X = X_ref[...]
    k = X @ wk_ref[...]
    q = M_ref[...] @ wq_ref[...]
    sim = lax.dot_general(q, k, (((1,), (1,)), ((), ()))) * SCALE
    g_iota = lax.broadcasted_iota(jnp.int32, (G, N), 0)
    mask = batch_ref[...] == g_iota
    sim = jnp.where(mask, sim, -1e9)
    mx = jnp.max(sim, axis=1, keepdims=True)
    e = jnp.exp(sim - mx)
    attn = e / jnp.sum(e, axis=1, keepdims=True)
    f_ref[...] = jnp.sum(attn, axis=0, keepdims=True) * LAMB
    xw_ref[...] = X @ gw_ref[...] + gb_ref[...]


def _tc_sim(X, batch2d, M, wq, wk, gw, gb):
    return pl.pallas_call(
        _sim_body,
        out_shape=(jax.ShapeDtypeStruct((1, N), jnp.float32),
                   jax.ShapeDtypeStruct((N, D), jnp.float32)),
    )(X, batch2d, M, wq, wk, gw, gb.reshape(1, -1))


def _head_body(M_ref, w1_ref, b1_ref, w2_ref, b2_ref, out_ref):
    h = jnp.maximum(M_ref[...] @ w1_ref[...] + b1_ref[...], 0.0)
    out_ref[...] = h @ w2_ref[...] + b2_ref[...]


def _tc_head(M, w1, b1, w2, b2):
    nc = w2.shape[1]
    return pl.pallas_call(
        _head_body,
        out_shape=jax.ShapeDtypeStruct((G, nc), jnp.float32),
    )(M, w1, b1.reshape(1, -1), w2, b2.reshape(1, -1))


def kernel(x, edge_index, batch, num_graphs, params):
    p = params
    ei = edge_index.astype(jnp.int32)
    loops = jnp.arange(N, dtype=jnp.int32)
    pad = EP - (ei.shape[1] + N)
    row_p = jnp.concatenate([ei[0], loops, jnp.full((pad,), N, jnp.int32)])
    col_p = jnp.concatenate([ei[1], loops, jnp.zeros((pad,), jnp.int32)])
    batch2d = batch.astype(jnp.int32).reshape(1, N)
    z128 = jnp.zeros((NP, 128), jnp.float32)

    deg2 = _sc_degcount_kernel()(row_p, z128)
    X = _tc_bn0(x, p['bn_feat_g'].reshape(1, -1), p['bn_feat_b'].reshape(1, -1))
    dis_flat = _tc_dis(deg2)[:, 0]

    M = jnp.tile(p['Memory'], (G, 1))
    for l in range(2):
        M = _tc_cross(X, batch2d, M, p['to_kv'][l], p)
        f1d, Xw = _tc_sim(X, batch2d, M, p['sim_q'], p['sim_k'][l],
                          p['gcn_W'][l], p['gcn_b'][l])
        xs = _sc_aggregate_kernel()(Xw, col_p, row_p, dis_flat,
                                    f1d.reshape(-1), z128)
        X = _tc_bnl(xs, p['bn_g'][l].reshape(1, -1), p['bn_b'][l].reshape(1, -1))

    M = _tc_cross(X, batch2d, M, p['to_kv'][2], p)
    return _tc_head(M, p['fc1_W'], p['fc1_b'], p['fc2_W'], p['fc2_b'])


# speedup vs baseline: 3.1084x; 1.0042x over previous
import functools

import jax
import jax.numpy as jnp
from jax import lax
from jax.experimental import pallas as pl
from jax.experimental.pallas import tpu as pltpu
from jax.experimental.pallas import tpu_sc as plsc

N = 10000
D = 128
G = 32
HEADS = 4
INNER = 64
KV = HEADS * INNER
LAMB = 0.5
SCALE = INNER ** -0.5

NP = 10240
EP = 331776
NWORK = 32
EPW = EP // NWORK
C = 128
NCHUNK = EPW // C
RPS = NP // 16


@functools.cache
def _sc_degcount_kernel():
    mesh = plsc.VectorSubcoreMesh(core_axis_name="c", subcore_axis_name="s")
    return functools.partial(
        pl.kernel,
        mesh=mesh,
        out_type=jax.ShapeDtypeStruct((2, NP, 128), jnp.float32),
        compiler_params=pltpu.CompilerParams(needs_layout_passes=False),
        scratch_types=[
            pltpu.VMEM((C,), jnp.int32),
            pltpu.VMEM((C, 128), jnp.float32),
            pltpu.VMEM_SHARED((NP, 128), jnp.float32),
        ],
    )(_sc_degcount_body)


def _sc_degcount_body(row_hbm, z128_hbm, out_hbm, row_v, ones_v, acc_sh):
    cid = lax.axis_index("c")
    sid = lax.axis_index("s")
    wid = sid * 2 + cid
    pltpu.sync_copy(z128_hbm.at[pl.ds(sid * RPS, RPS)],
                    acc_sh.at[pl.ds(sid * RPS, RPS)])

    def fill(i, carry):
        for j in range(8):
            ones_v[i, pl.ds(j * 16, 16)] = jnp.full((16,), 1.0, jnp.float32)
        return carry

    lax.fori_loop(0, C, fill, 0)
    plsc.subcore_barrier()

    def body(i, carry):
        off = pl.multiple_of(wid * EPW + i * C, 8)
        pltpu.sync_copy(row_hbm.at[pl.ds(off, C)], row_v)
        pltpu.sync_copy(ones_v, acc_sh.at[row_v], add=True)
        return carry

    lax.fori_loop(0, NCHUNK, body, 0)
    plsc.subcore_barrier()
    pltpu.sync_copy(acc_sh.at[pl.ds(sid * RPS, RPS)],
                    out_hbm.at[cid, pl.ds(sid * RPS, RPS)])


@functools.cache
def _sc_aggregate_kernel():
    mesh = plsc.VectorSubcoreMesh(core_axis_name="c", subcore_axis_name="s")
    return functools.partial(
        pl.kernel,
        mesh=mesh,
        out_type=jax.ShapeDtypeStruct((2, NP, 128), jnp.float32),
        compiler_params=pltpu.CompilerParams(needs_layout_passes=False),
        scratch_types=[
            pltpu.VMEM((C,), jnp.int32),
            pltpu.VMEM((C,), jnp.int32),
            pltpu.VMEM((C,), jnp.float32),
            pltpu.VMEM((C, 128), jnp.float32),
            pltpu.VMEM((NP,), jnp.float32),
            pltpu.VMEM((NP,), jnp.float32),
            pltpu.VMEM_SHARED((NP, 128), jnp.float32),
            pltpu.SemaphoreType.DMA,
        ],
    )(_sc_aggregate_body)


def _sc_aggregate_body(xw_hbm, col_hbm, row_hbm, dis_hbm, f_hbm, z128_hbm, out_hbm,
                       col_v, row_v, w_v, rows_v, dis_v, f_v, acc_sh, sem):
    cid = lax.axis_index("c")
    sid = lax.axis_index("s")
    wid = sid * 2 + cid
    pltpu.sync_copy(z128_hbm.at[pl.ds(sid * RPS, RPS)],
                    acc_sh.at[pl.ds(sid * RPS, RPS)])
    pltpu.sync_copy(dis_hbm, dis_v)
    pltpu.sync_copy(f_hbm, f_v.at[pl.ds(0, N)])
    plsc.subcore_barrier()

    def body(i, carry):
        off = pl.multiple_of(wid * EPW + i * C, 8)
        pltpu.sync_copy(col_hbm.at[pl.ds(off, C)], col_v)
        pltpu.sync_copy(row_hbm.at[pl.ds(off, C)], row_v)
        pltpu.async_copy(xw_hbm.at[col_v], rows_v, sem).wait()
        for i16 in range(C // 16):
            rv = row_v[pl.ds(i16 * 16, 16)]
            cv = col_v[pl.ds(i16 * 16, 16)]
            dr = plsc.load_gather(dis_v, [rv])
            dc = plsc.load_gather(dis_v, [cv])
            fr = plsc.load_gather(f_v, [rv])
            w_v[pl.ds(i16 * 16, 16)] = dr * dc + fr

        def scale(e8, c2):
            e = e8 * 8
            ws = [plsc.load_gather(w_v, [jnp.full((16,), d, jnp.int32) + e])
                  for d in range(8)]
            for j in range(8):
                for d in range(8):
                    rows_v[e + d, pl.ds(j * 16, 16)] = \
                        rows_v[e + d, pl.ds(j * 16, 16)] * ws[d]
            return c2

        lax.fori_loop(0, C // 8, scale, 0)
        pltpu.sync_copy(rows_v, acc_sh.at[row_v], add=True)
        return carry

    lax.fori_loop(0, NCHUNK, body, 0)
    plsc.subcore_barrier()
    pltpu.sync_copy(acc_sh.at[pl.ds(sid * RPS, RPS)],
                    out_hbm.at[cid, pl.ds(sid * RPS, RPS)])


def _bn0_body(x_ref, g_ref, b_ref, X_ref):
    x = x_ref[...]
    m = jnp.mean(x, axis=0)
    v = jnp.mean((x - m) ** 2, axis=0)
    X_ref[...] = (x - m) * lax.rsqrt(v + 1e-5) * g_ref[...] + b_ref[...]


def _tc_bn0(x, g, b):
    return pl.pallas_call(
        _bn0_body,
        out_shape=jax.ShapeDtypeStruct((N, D), jnp.float32),
    )(x, g, b)


def _dis_body(deg_ref, dis_ref):
    deg = deg_ref[0, :, :16] + deg_ref[1, :, :16]
    dis_ref[...] = jnp.where(deg > 0.0, lax.rsqrt(jnp.maximum(deg, 1e-30)), 0.0)


def _tc_dis(deg2):
    return pl.pallas_call(
        _dis_body,
        out_shape=jax.ShapeDtypeStruct((NP, 16), jnp.float32),
    )(deg2)


def _bnl_body(xs_ref, g_ref, b_ref, X_ref):
    xsum = xs_ref[0, :N, :] + xs_ref[1, :N, :]
    x = jnp.maximum(xsum, 0.0)
    m = jnp.mean(x, axis=0)
    v = jnp.mean((x - m) ** 2, axis=0)
    X_ref[...] = (x - m) * lax.rsqrt(v + 1e-5) * g_ref[...] + b_ref[...]


def _tc_bnl(xs, g, b):
    return pl.pallas_call(
        _bnl_body,
        out_shape=jax.ShapeDtypeStruct((N, D), jnp.float32),
    )(xs, g, b)


def _cross_body(X_ref, batch_ref, M_ref, kw_ref, vw_ref, qw_ref, wo_ref,
                w1_ref, b1_ref, w2_ref, b2_ref, acc_ref, Mn_ref):
    h = pl.program_id(0)
    X = X_ref[...]
    k = X @ kw_ref[0]
    v = X @ vw_ref[0]
    qh = M_ref[...] @ qw_ref[0]
    sim = lax.dot_general(qh, k, (((1,), (1,)), ((), ()))) * SCALE
    g_iota = lax.broadcasted_iota(jnp.int32, (G, N), 0)
    mask = batch_ref[...] == g_iota
    sim = jnp.where(mask, sim, -1e9)
    mx = jnp.max(sim, axis=1, keepdims=True)
    e = jnp.exp(sim - mx)
    attn = e / jnp.sum(e, axis=1, keepdims=True)
    contrib = (attn @ v) @ wo_ref[0]

    @pl.when(h == 0)
    def _():
        acc_ref[...] = contrib

    @pl.when(h > 0)
    def _():
        acc_ref[...] = acc_ref[...] + contrib

    @pl.when(h == HEADS - 1)
    def _():
        M = M_ref[...] + acc_ref[...]
        M = M + jnp.maximum(M @ w1_ref[...] + b1_ref[...], 0.0) @ w2_ref[...] \
            + b2_ref[...]
        Mn_ref[...] = M


def _tc_cross(X, batch2d, M, tokv, p):
    full = lambda s: pl.BlockSpec(s, lambda h: (0,) * len(s))
    head = lambda s: pl.BlockSpec((1,) + s, lambda h: (h, 0, 0))
    kw4 = tokv[:, :KV].reshape(D, HEADS, INNER).transpose(1, 0, 2)
    vw4 = tokv[:, KV:].reshape(D, HEADS, INNER).transpose(1, 0, 2)
    qw4 = p['Wq'].reshape(D, HEADS, INNER).transpose(1, 0, 2)
    wo4 = p['Wo'].reshape(HEADS, INNER, D)
    acc, Mn = pl.pallas_call(
        _cross_body,
        grid=(HEADS,),
        in_specs=[full((N, D)), full((1, N)), full((G, D)),
                  head((D, INNER)), head((D, INNER)), head((D, INNER)),
                  head((INNER, D)), full((D, 2 * D)),
                  full((1, 2 * D)), full((2 * D, D)), full((1, D))],
        out_specs=(full((G, D)), full((G, D))),
        out_shape=(jax.ShapeDtypeStruct((G, D), jnp.float32),
                   jax.ShapeDtypeStruct((G, D), jnp.float32)),
    )(X, batch2d, M, kw4, vw4, qw4, wo4, p['W1'],
      p['b1'].reshape(1, -1), p['W2'], p['b2'].reshape(1, -1))
    return Mn


def _sim_body(X_ref, batch_ref, M_ref, wq_ref, wk_ref, gw_ref, gb_ref,
              f_ref, xw_ref):
    X = X_ref[...]
    k = X @ wk_ref[...]
    q = M_ref[...] @ wq_ref[...]
    sim = lax.dot_general(q, k, (((1,), (1,)), ((), ()))) * SCALE
    g_iota = lax.broadcasted_iota(jnp.int32, (G, N), 0)
    mask = batch_ref[...] == g_iota
    sim = jnp.where(mask, sim, -1e9)
    mx = jnp.max(sim, axis=1, keepdims=True)
    e = jnp.exp(sim - mx)
    attn = e / jnp.sum(e, axis=1, keepdims=True)
    f_ref[...] = jnp.sum(attn, axis=0, keepdims=True) * LAMB
    xw_ref[...] = X @ gw_ref[...] + gb_ref[...]


def _tc_sim(X, batch2d, M, wq, wk, gw, gb):
    return pl.pallas_call(
        _sim_body,
        out_shape=(jax.ShapeDtypeStruct((1, N), jnp.float32),
                   jax.ShapeDtypeStruct((N, D), jnp.float32)),
    )(X, batch2d, M, wq, wk, gw, gb.reshape(1, -1))


def _head_body(M_ref, w1_ref, b1_ref, w2_ref, b2_ref, out_ref):
    h = jnp.maximum(M_ref[...] @ w1_ref[...] + b1_ref[...], 0.0)
    out_ref[...] = h @ w2_ref[...] + b2_ref[...]


def _tc_head(M, w1, b1, w2, b2):
    nc = w2.shape[1]
    return pl.pallas_call(
        _head_body,
        out_shape=jax.ShapeDtypeStruct((G, nc), jnp.float32),
    )(M, w1, b1.reshape(1, -1), w2, b2.reshape(1, -1))


def kernel(x, edge_index, batch, num_graphs, params):
    p = params
    ei = edge_index.astype(jnp.int32)
    loops = jnp.arange(N, dtype=jnp.int32)
    pad = EP - (ei.shape[1] + N)
    row_p = jnp.concatenate([ei[0], loops, jnp.full((pad,), N, jnp.int32)])
    col_p = jnp.concatenate([ei[1], loops, jnp.zeros((pad,), jnp.int32)])
    batch2d = batch.astype(jnp.int32).reshape(1, N)
    z128 = jnp.zeros((NP, 128), jnp.float32)

    deg2 = _sc_degcount_kernel()(row_p, z128)
    X = _tc_bn0(x, p['bn_feat_g'].reshape(1, -1), p['bn_feat_b'].reshape(1, -1))
    dis_flat = _tc_dis(deg2)[:, 0]

    M = jnp.tile(p['Memory'], (G, 1))
    for l in range(2):
        M = _tc_cross(X, batch2d, M, p['to_kv'][l], p)
        f1d, Xw = _tc_sim(X, batch2d, M, p['sim_q'], p['sim_k'][l],
                          p['gcn_W'][l], p['gcn_b'][l])
        xs = _sc_aggregate_kernel()(Xw, col_p, row_p, dis_flat,
                                    f1d.reshape(-1), z128)
        X = _tc_bnl(xs, p['bn_g'][l].reshape(1, -1), p['bn_b'][l].reshape(1, -1))

    M = _tc_cross(X, batch2d, M, p['to_kv'][2], p)
    return _tc_head(M, p['fc1_W'], p['fc1_b'], p['fc2_W'], p['fc2_b'])
